# baseline (speedup 1.0000x reference)
import functools

import jax
import jax.numpy as jnp
from jax import lax
from jax.experimental import pallas as pl
from jax.experimental.pallas import tpu as pltpu
from jax.experimental.pallas import tpu_sc as plsc

N = 10000
E = 320000
D = 128

NPAD = 10240
EPAD = 327680
CHUNK = 128
NW = 32
CPW = EPAD // (NW * CHUNK)
EPW = CPW * CHUNK
ROWS_PER_TILE = NPAD // 16


def _make_sc_agg(with_cnt):
    mesh = plsc.VectorSubcoreMesh(core_axis_name="c", subcore_axis_name="s")
    out_type = [jax.ShapeDtypeStruct((2, NPAD, D), jnp.float32)]
    scratch = [
        pltpu.VMEM((3, 2, CHUNK), jnp.int32),
        pltpu.VMEM((2, CHUNK, D), jnp.float32),
        pltpu.VMEM((CHUNK,), jnp.float32),
        pltpu.VMEM_SHARED((NPAD, D), jnp.float32),
        pltpu.VMEM_SHARED((NPAD,), jnp.float32),
        pltpu.SemaphoreType.DMA,
        pltpu.SemaphoreType.DMA,
        pltpu.SemaphoreType.DMA,
    ]
    if with_cnt:
        out_type = out_type + [jax.ShapeDtypeStruct((2, NPAD), jnp.float32)]

    @functools.partial(pl.kernel, mesh=mesh, out_type=out_type,
                       scratch_types=scratch)
    def agg(h_hbm, e_hbm, *rest):
        if with_cnt:
            (sums_out, cnt_out,
             idx, rows, ones_v, acc, acc_cnt, sem_i, sem_g, sem_s) = rest
        else:
            (sums_out,
             idx, rows, ones_v, acc, acc_cnt, sem_i, sem_g, sem_s) = rest
        c = lax.axis_index("c")
        s = lax.axis_index("s")
        w = c * 16 + s
        g0 = w * CPW

        zeros16 = jnp.zeros((16,), jnp.float32)
        ones16 = jnp.ones((16,), jnp.float32)

        def zrow(r, carry):
            for kk in range(8):
                rows[0, r, pl.ds(kk * 16, 16)] = zeros16
            return carry
        lax.fori_loop(0, CHUNK, zrow, 0)
        if with_cnt:
            for kk in range(CHUNK // 16):
                ones_v[pl.ds(kk * 16, 16)] = ones16

        slab = s * ROWS_PER_TILE
        for t in range(ROWS_PER_TILE // CHUNK):
            pltpu.sync_copy(rows.at[0], acc.at[pl.ds(slab + t * CHUNK, CHUNK)])
        if with_cnt:
            for t in range(ROWS_PER_TILE // CHUNK):
                pltpu.sync_copy(rows.at[0, 0],
                                acc_cnt.at[pl.ds(slab + t * CHUNK, CHUNK)])

        plsc.subcore_barrier()

        pltpu.sync_copy(e_hbm.at[g0], idx.at[0])
        pltpu.async_copy(h_hbm.at[idx.at[0, 0]], rows.at[0], sem_g)
        pltpu.async_copy(e_hbm.at[g0 + 1], idx.at[1], sem_i)

        def step(j, b):
            ib = lax.rem(j, 3)
            pltpu.make_async_copy(
                h_hbm.at[idx.at[0, 0]], rows.at[b], sem_g).wait()

            @pl.when(j >= 1)
            def _():
                pltpu.make_async_copy(
                    rows.at[0], acc.at[idx.at[0, 1]], sem_s).wait()
                if with_cnt:
                    pltpu.make_async_copy(
                        ones_v, acc_cnt.at[idx.at[0, 1]], sem_s).wait()

            @pl.when(j + 1 < CPW)
            def _():
                pltpu.make_async_copy(
                    e_hbm.at[g0], idx.at[0], sem_i).wait()
                pltpu.async_copy(
                    h_hbm.at[idx.at[lax.rem(j + 1, 3), 0]],
                    rows.at[1 - b], sem_g)

            pltpu.async_copy(rows.at[b], acc.at[idx.at[ib, 1]], sem_s,
                             add=True)
            if with_cnt:
                pltpu.async_copy(ones_v, acc_cnt.at[idx.at[ib, 1]], sem_s,
                                 add=True)

            @pl.when(j + 2 < CPW)
            def _():
                pltpu.async_copy(e_hbm.at[g0 + j + 2],
                                 idx.at[lax.rem(j + 2, 3)], sem_i)

        def chunk_pair(jj, carry):
            step(jj * 2, 0)
            step(jj * 2 + 1, 1)
            return carry
        lax.fori_loop(0, CPW // 2, chunk_pair, 0)

        pltpu.make_async_copy(rows.at[0], acc.at[idx.at[0, 1]], sem_s).wait()
        if with_cnt:
            pltpu.make_async_copy(
                ones_v, acc_cnt.at[idx.at[0, 1]], sem_s).wait()

        plsc.subcore_barrier()

        for t in range(ROWS_PER_TILE // CHUNK):
            pltpu.sync_copy(
                acc.at[pl.ds(slab + t * CHUNK, CHUNK)],
                sums_out.at[c, pl.ds(slab + t * CHUNK, CHUNK)])
        if with_cnt:
            pltpu.sync_copy(acc_cnt.at[pl.ds(slab, ROWS_PER_TILE)],
                            cnt_out.at[c, pl.ds(slab, ROWS_PER_TILE)])

    return agg


_sc_agg_cnt = _make_sc_agg(True)
_sc_agg_nocnt = _make_sc_agg(False)


def _tc_layer(h_pad, sums_p, cnt_t, wa, wb, b2d):
    blk = 1024

    def body(h_ref, s_ref, c_ref, wa_ref, wb_ref, b_ref, o_ref):
        sums = s_ref[0] + s_ref[1]
        cnt = jnp.sum(c_ref[...], axis=1, keepdims=True)
        mean = sums * (1.0 / jnp.maximum(cnt, 1.0))
        acc = jnp.dot(h_ref[...], wa_ref[...], preferred_element_type=jnp.float32)
        acc = acc + jnp.dot(mean, wb_ref[...], preferred_element_type=jnp.float32)
        o_ref[...] = jnp.maximum(acc + b_ref[...], 0.0)

    return pl.pallas_call(
        body,
        grid=(NPAD // blk,),
        in_specs=[
            pl.BlockSpec((blk, D), lambda i: (i, 0)),
            pl.BlockSpec((2, blk, D), lambda i: (0, i, 0)),
            pl.BlockSpec((blk, 2), lambda i: (i, 0)),
            pl.BlockSpec((D, D), lambda i: (0, 0)),
            pl.BlockSpec((D, D), lambda i: (0, 0)),
            pl.BlockSpec((1, D), lambda i: (0, 0)),
        ],
        out_specs=pl.BlockSpec((blk, D), lambda i: (i, 0)),
        out_shape=jax.ShapeDtypeStruct((NPAD, D), jnp.float32),
    )(h_pad, sums_p, cnt_t, wa, wb, b2d)


def kernel(h, edge_index, W1, b1, W2, b2):
    src = edge_index[0].astype(jnp.int32)
    dst = edge_index[1].astype(jnp.int32)

    epad = EPAD - E
    pad_ar = jnp.arange(epad, dtype=jnp.int32)
    src_p = jnp.concatenate([src, (pad_ar * 37) % NPAD])
    dst_p = jnp.concatenate([dst, N + pad_ar % (NPAD - N)])
    edges3d = jnp.stack(
        [src_p.reshape(EPAD // CHUNK, CHUNK),
         dst_p.reshape(EPAD // CHUNK, CHUNK)], axis=1)
    h_p = jnp.pad(h, ((0, NPAD - N), (0, 0)))

    sums_p, cnt_all = _sc_agg_cnt(h_p, edges3d)
    cnt_t = cnt_all.T
    h_p = _tc_layer(h_p, sums_p, cnt_t, W1[:D], W1[D:], b1.reshape(1, D))
    (sums_p2,) = _sc_agg_nocnt(h_p, edges3d)
    h_p = _tc_layer(h_p, sums_p2, cnt_t, W2[:D], W2[D:], b2.reshape(1, D))
    return h_p[:N]


# speedup vs baseline: 1.0215x; 1.0215x over previous
import functools

import jax
import jax.numpy as jnp
from jax import lax
from jax.experimental import pallas as pl
from jax.experimental.pallas import tpu as pltpu
from jax.experimental.pallas import tpu_sc as plsc

N = 10000
E = 320000
D = 128

NPAD = 10240
EPAD = 327680
CHUNK = 128
NW = 32
CPW = EPAD // (NW * CHUNK)
EPW = CPW * CHUNK
ROWS_PER_TILE = NPAD // 16


def _make_sc_agg(with_cnt):
    mesh = plsc.VectorSubcoreMesh(core_axis_name="c", subcore_axis_name="s")
    out_type = [jax.ShapeDtypeStruct((2, NPAD, D), jnp.float32)]
    scratch = [
        pltpu.VMEM((3, 2, CHUNK), jnp.int32),
        pltpu.VMEM((2, CHUNK, D), jnp.float32),
        pltpu.VMEM((CHUNK,), jnp.float32),
        pltpu.VMEM_SHARED((NPAD, D), jnp.float32),
        pltpu.VMEM_SHARED((NPAD,), jnp.float32),
        pltpu.SemaphoreType.DMA,
        pltpu.SemaphoreType.DMA,
        pltpu.SemaphoreType.DMA,
    ]
    if with_cnt:
        out_type = out_type + [jax.ShapeDtypeStruct((2, NPAD), jnp.float32)]

    @functools.partial(pl.kernel, mesh=mesh, out_type=out_type,
                       scratch_types=scratch)
    def agg(h_hbm, e_hbm, *rest):
        if with_cnt:
            (sums_out, cnt_out,
             idx, rows, ones_v, acc, acc_cnt, sem_i, sem_g, sem_s) = rest
        else:
            (sums_out,
             idx, rows, ones_v, acc, acc_cnt, sem_i, sem_g, sem_s) = rest
        c = lax.axis_index("c")
        s = lax.axis_index("s")
        w = c * 16 + s
        g0 = w * CPW

        zeros16 = jnp.zeros((16,), jnp.float32)
        ones16 = jnp.ones((16,), jnp.float32)

        def zrow(r, carry):
            for kk in range(8):
                rows[0, r, pl.ds(kk * 16, 16)] = zeros16
            return carry
        lax.fori_loop(0, CHUNK, zrow, 0)
        if with_cnt:
            for kk in range(CHUNK // 16):
                ones_v[pl.ds(kk * 16, 16)] = ones16

        slab = s * ROWS_PER_TILE
        for t in range(ROWS_PER_TILE // CHUNK):
            pltpu.async_copy(rows.at[0], acc.at[pl.ds(slab + t * CHUNK, CHUNK)],
                             sem_s)
        if with_cnt:
            for t in range(ROWS_PER_TILE // CHUNK):
                pltpu.async_copy(rows.at[0, 0],
                                 acc_cnt.at[pl.ds(slab + t * CHUNK, CHUNK)],
                                 sem_s)
        for t in range(ROWS_PER_TILE // CHUNK):
            pltpu.make_async_copy(
                rows.at[0], acc.at[pl.ds(slab, CHUNK)], sem_s).wait()
            if with_cnt:
                pltpu.make_async_copy(
                    rows.at[0, 0], acc_cnt.at[pl.ds(slab, CHUNK)], sem_s).wait()

        plsc.subcore_barrier()

        pltpu.sync_copy(e_hbm.at[g0], idx.at[0])
        pltpu.async_copy(h_hbm.at[idx.at[0, 0]], rows.at[0], sem_g)
        pltpu.async_copy(e_hbm.at[g0 + 1], idx.at[1], sem_i)

        def step(j, b):
            ib = lax.rem(j, 3)
            pltpu.make_async_copy(
                h_hbm.at[idx.at[0, 0]], rows.at[b], sem_g).wait()

            @pl.when(j >= 1)
            def _():
                pltpu.make_async_copy(
                    rows.at[0], acc.at[idx.at[0, 1]], sem_s).wait()
                if with_cnt:
                    pltpu.make_async_copy(
                        ones_v, acc_cnt.at[idx.at[0, 1]], sem_s).wait()

            @pl.when(j + 1 < CPW)
            def _():
                pltpu.make_async_copy(
                    e_hbm.at[g0], idx.at[0], sem_i).wait()
                pltpu.async_copy(
                    h_hbm.at[idx.at[lax.rem(j + 1, 3), 0]],
                    rows.at[1 - b], sem_g)

            pltpu.async_copy(rows.at[b], acc.at[idx.at[ib, 1]], sem_s,
                             add=True)
            if with_cnt:
                pltpu.async_copy(ones_v, acc_cnt.at[idx.at[ib, 1]], sem_s,
                                 add=True)

            @pl.when(j + 2 < CPW)
            def _():
                pltpu.async_copy(e_hbm.at[g0 + j + 2],
                                 idx.at[lax.rem(j + 2, 3)], sem_i)

        def chunk_pair(jj, carry):
            step(jj * 2, 0)
            step(jj * 2 + 1, 1)
            return carry
        lax.fori_loop(0, CPW // 2, chunk_pair, 0)

        pltpu.make_async_copy(rows.at[0], acc.at[idx.at[0, 1]], sem_s).wait()
        if with_cnt:
            pltpu.make_async_copy(
                ones_v, acc_cnt.at[idx.at[0, 1]], sem_s).wait()

        plsc.subcore_barrier()

        pltpu.sync_copy(acc.at[pl.ds(slab, ROWS_PER_TILE)],
                        sums_out.at[c, pl.ds(slab, ROWS_PER_TILE)])
        if with_cnt:
            pltpu.sync_copy(acc_cnt.at[pl.ds(slab, ROWS_PER_TILE)],
                            cnt_out.at[c, pl.ds(slab, ROWS_PER_TILE)])

    return agg


_sc_agg_cnt = _make_sc_agg(True)
_sc_agg_nocnt = _make_sc_agg(False)


def _tc_layer(h, sums_p, cnt_t, wa, wb, b2d):
    blk = 1000

    def body(h_ref, s_ref, c_ref, wa_ref, wb_ref, b_ref, o_ref):
        sums = s_ref[0] + s_ref[1]
        cnt = jnp.sum(c_ref[...], axis=1, keepdims=True)
        mean = sums * (1.0 / jnp.maximum(cnt, 1.0))
        acc = jnp.dot(h_ref[...], wa_ref[...], preferred_element_type=jnp.float32)
        acc = acc + jnp.dot(mean, wb_ref[...], preferred_element_type=jnp.float32)
        o_ref[...] = jnp.maximum(acc + b_ref[...], 0.0)

    return pl.pallas_call(
        body,
        grid=(N // blk,),
        in_specs=[
            pl.BlockSpec((blk, D), lambda i: (i, 0)),
            pl.BlockSpec((2, blk, D), lambda i: (0, i, 0)),
            pl.BlockSpec((blk, 2), lambda i: (i, 0)),
            pl.BlockSpec((D, D), lambda i: (0, 0)),
            pl.BlockSpec((D, D), lambda i: (0, 0)),
            pl.BlockSpec((1, D), lambda i: (0, 0)),
        ],
        out_specs=pl.BlockSpec((blk, D), lambda i: (i, 0)),
        out_shape=jax.ShapeDtypeStruct((N, D), jnp.float32),
    )(h, sums_p, cnt_t, wa, wb, b2d)


def kernel(h, edge_index, W1, b1, W2, b2):
    src = edge_index[0].astype(jnp.int32)
    dst = edge_index[1].astype(jnp.int32)

    epad = EPAD - E
    pad_ar = jnp.arange(epad, dtype=jnp.int32)
    src_p = jnp.concatenate([src, (pad_ar * 37) % N])
    dst_p = jnp.concatenate([dst, N + pad_ar % (NPAD - N)])
    edges3d = jnp.stack(
        [src_p.reshape(EPAD // CHUNK, CHUNK),
         dst_p.reshape(EPAD // CHUNK, CHUNK)], axis=1)

    sums_p, cnt_all = _sc_agg_cnt(h, edges3d)
    cnt_t = cnt_all.T
    h = _tc_layer(h, sums_p, cnt_t, W1[:D], W1[D:], b1.reshape(1, D))
    (sums_p2,) = _sc_agg_nocnt(h, edges3d)
    return _tc_layer(h, sums_p2, cnt_t, W2[:D], W2[D:], b2.reshape(1, D))


# speedup vs baseline: 1.1849x; 1.1599x over previous
import functools

import jax
import jax.numpy as jnp
from jax import lax
from jax.experimental import pallas as pl
from jax.experimental.pallas import tpu as pltpu
from jax.experimental.pallas import tpu_sc as plsc

N = 10000
E = 320000
D = 128

NPAD = 10240
EPAD = 327680
CHUNK = 128
NW = 32
CPW = EPAD // (NW * CHUNK)
EPW = CPW * CHUNK
ROWS_PER_TILE = NPAD // 16


def _make_sc_agg(with_cnt, with_rows=True):
    mesh = plsc.VectorSubcoreMesh(core_axis_name="c", subcore_axis_name="s")
    out_type = [jax.ShapeDtypeStruct((2, NPAD, D), jnp.float32)]
    scratch = [
        pltpu.VMEM((3, 2, CHUNK), jnp.int32),
        pltpu.VMEM((2, CHUNK, D), jnp.float32),
        pltpu.VMEM((CHUNK,), jnp.float32),
        pltpu.VMEM_SHARED((NPAD, D), jnp.float32),
        pltpu.VMEM_SHARED((NPAD,), jnp.float32),
        pltpu.SemaphoreType.DMA,
        pltpu.SemaphoreType.DMA,
        pltpu.SemaphoreType.DMA,
    ]
    if with_cnt:
        out_type = out_type + [jax.ShapeDtypeStruct((2, NPAD), jnp.float32)]

    @functools.partial(pl.kernel, mesh=mesh, out_type=out_type,
                       scratch_types=scratch)
    def agg(h_hbm, e_hbm, *rest):
        if with_cnt:
            (sums_out, cnt_out,
             idx, rows, ones_v, acc, acc_cnt, sem_i, sem_g, sem_s) = rest
        else:
            (sums_out,
             idx, rows, ones_v, acc, acc_cnt, sem_i, sem_g, sem_s) = rest
        c = lax.axis_index("c")
        s = lax.axis_index("s")
        w = c * 16 + s
        g0 = w * CPW

        zeros16 = jnp.zeros((16,), jnp.float32)
        ones16 = jnp.ones((16,), jnp.float32)

        def zrow(r, carry):
            for kk in range(8):
                rows[0, r, pl.ds(kk * 16, 16)] = zeros16
            return carry
        lax.fori_loop(0, CHUNK, zrow, 0)
        if with_cnt:
            for kk in range(CHUNK // 16):
                ones_v[pl.ds(kk * 16, 16)] = ones16

        slab = s * ROWS_PER_TILE
        for t in range(ROWS_PER_TILE // CHUNK):
            pltpu.async_copy(rows.at[0], acc.at[pl.ds(slab + t * CHUNK, CHUNK)],
                             sem_s)
        if with_cnt:
            for t in range(ROWS_PER_TILE // CHUNK):
                pltpu.async_copy(rows.at[0, 0],
                                 acc_cnt.at[pl.ds(slab + t * CHUNK, CHUNK)],
                                 sem_s)
        for t in range(ROWS_PER_TILE // CHUNK):
            pltpu.make_async_copy(
                rows.at[0], acc.at[pl.ds(slab, CHUNK)], sem_s).wait()
            if with_cnt:
                pltpu.make_async_copy(
                    rows.at[0, 0], acc_cnt.at[pl.ds(slab, CHUNK)], sem_s).wait()

        plsc.subcore_barrier()

        pltpu.sync_copy(e_hbm.at[g0], idx.at[0])
        pltpu.async_copy(h_hbm.at[idx.at[0, 0]], rows.at[0], sem_g)
        pltpu.async_copy(e_hbm.at[g0 + 1], idx.at[1], sem_i)

        def step(j, b):
            ib = lax.rem(j, 3)

            @pl.when(j >= 1)
            def _():
                if with_rows:
                    pltpu.make_async_copy(
                        rows.at[0], acc.at[idx.at[0, 1]], sem_s).wait()
                if with_cnt:
                    pltpu.make_async_copy(
                        ones_v, acc_cnt.at[idx.at[0, 1]], sem_s).wait()

            @pl.when(j + 1 < CPW)
            def _():
                pltpu.make_async_copy(
                    e_hbm.at[g0], idx.at[0], sem_i).wait()
                pltpu.async_copy(
                    h_hbm.at[idx.at[lax.rem(j + 1, 3), 0]],
                    rows.at[1 - b], sem_g)

            pltpu.make_async_copy(
                h_hbm.at[idx.at[0, 0]], rows.at[b], sem_g).wait()
            if with_rows:
                pltpu.async_copy(rows.at[b], acc.at[idx.at[ib, 1]], sem_s,
                                 add=True)
            if with_cnt:
                pltpu.async_copy(ones_v, acc_cnt.at[idx.at[ib, 1]], sem_s,
                                 add=True)

            @pl.when(j + 2 < CPW)
            def _():
                pltpu.async_copy(e_hbm.at[g0 + j + 2],
                                 idx.at[lax.rem(j + 2, 3)], sem_i)

        def chunk_pair(jj, carry):
            step(jj * 2, 0)
            step(jj * 2 + 1, 1)
            return carry
        lax.fori_loop(0, CPW // 2, chunk_pair, 0)

        if with_rows:
            pltpu.make_async_copy(
                rows.at[0], acc.at[idx.at[0, 1]], sem_s).wait()
        if with_cnt:
            pltpu.make_async_copy(
                ones_v, acc_cnt.at[idx.at[0, 1]], sem_s).wait()

        plsc.subcore_barrier()

        pltpu.sync_copy(acc.at[pl.ds(slab, ROWS_PER_TILE)],
                        sums_out.at[c, pl.ds(slab, ROWS_PER_TILE)])
        if with_cnt:
            pltpu.sync_copy(acc_cnt.at[pl.ds(slab, ROWS_PER_TILE)],
                            cnt_out.at[c, pl.ds(slab, ROWS_PER_TILE)])

    return agg


_sc_agg_cnt = _make_sc_agg(True)
_sc_agg_nocnt = _make_sc_agg(False)


def _tc_layer(h, sums_p, cnt_t, wa, wb, b2d):
    blk = 1000

    def body(h_ref, s_ref, c_ref, wa_ref, wb_ref, b_ref, o_ref):
        sums = s_ref[0] + s_ref[1]
        cnt = jnp.sum(c_ref[...], axis=1, keepdims=True)
        mean = sums * (1.0 / jnp.maximum(cnt, 1.0))
        acc = jnp.dot(h_ref[...], wa_ref[...], preferred_element_type=jnp.float32)
        acc = acc + jnp.dot(mean, wb_ref[...], preferred_element_type=jnp.float32)
        o_ref[...] = jnp.maximum(acc + b_ref[...], 0.0)

    return pl.pallas_call(
        body,
        grid=(N // blk,),
        in_specs=[
            pl.BlockSpec((blk, D), lambda i: (i, 0)),
            pl.BlockSpec((2, blk, D), lambda i: (0, i, 0)),
            pl.BlockSpec((blk, 2), lambda i: (i, 0)),
            pl.BlockSpec((D, D), lambda i: (0, 0)),
            pl.BlockSpec((D, D), lambda i: (0, 0)),
            pl.BlockSpec((1, D), lambda i: (0, 0)),
        ],
        out_specs=pl.BlockSpec((blk, D), lambda i: (i, 0)),
        out_shape=jax.ShapeDtypeStruct((N, D), jnp.float32),
    )(h, sums_p, cnt_t, wa, wb, b2d)


def kernel(h, edge_index, W1, b1, W2, b2):
    src = edge_index[0].astype(jnp.int32)
    dst = edge_index[1].astype(jnp.int32)

    epad = EPAD - E
    pad_ar = jnp.arange(epad, dtype=jnp.int32)
    src_p = jnp.concatenate([src, (pad_ar * 37) % N])
    dst_p = jnp.concatenate([dst, N + pad_ar % (NPAD - N)])
    edges3d = jnp.stack(
        [src_p.reshape(EPAD // CHUNK, CHUNK),
         dst_p.reshape(EPAD // CHUNK, CHUNK)], axis=1)

    sums_p, cnt_all = _sc_agg_cnt(h, edges3d)
    cnt_t = cnt_all.T
    h = _tc_layer(h, sums_p, cnt_t, W1[:D], W1[D:], b1.reshape(1, D))
    (sums_p2,) = _sc_agg_nocnt(h, edges3d)
    return _tc_layer(h, sums_p2, cnt_t, W2[:D], W2[D:], b2.reshape(1, D))


# speedup vs baseline: 1.2817x; 1.0817x over previous
import functools

import jax
import jax.numpy as jnp
from jax import lax
from jax.experimental import pallas as pl
from jax.experimental.pallas import tpu as pltpu
from jax.experimental.pallas import tpu_sc as plsc

N = 10000
E = 320000
D = 128

NPAD = 10240
EPAD = 327680
CHUNK = 64
NW = 32
CPW = EPAD // (NW * CHUNK)
EPW = CPW * CHUNK
ROWS_PER_TILE = NPAD // 16
RB = 5
IB = 7


def _make_sc_agg(with_cnt, with_rows=True):
    mesh = plsc.VectorSubcoreMesh(core_axis_name="c", subcore_axis_name="s")
    out_type = [jax.ShapeDtypeStruct((2, NPAD, D), jnp.float32)]
    scratch = [
        pltpu.VMEM((IB, 2, CHUNK), jnp.int32),
        pltpu.VMEM((RB, CHUNK, D), jnp.float32),
        pltpu.VMEM((CHUNK,), jnp.float32),
        pltpu.VMEM_SHARED((NPAD, D), jnp.float32),
        pltpu.VMEM_SHARED((NPAD,), jnp.float32),
        pltpu.SemaphoreType.DMA,
        pltpu.SemaphoreType.DMA,
        pltpu.SemaphoreType.DMA,
    ]
    if with_cnt:
        out_type = out_type + [jax.ShapeDtypeStruct((2, NPAD), jnp.float32)]

    @functools.partial(pl.kernel, mesh=mesh, out_type=out_type,
                       scratch_types=scratch)
    def agg(h_hbm, e_hbm, *rest):
        if with_cnt:
            (sums_out, cnt_out,
             idx, rows, ones_v, acc, acc_cnt, sem_i, sem_g, sem_s) = rest
        else:
            (sums_out,
             idx, rows, ones_v, acc, acc_cnt, sem_i, sem_g, sem_s) = rest
        c = lax.axis_index("c")
        s = lax.axis_index("s")
        w = c * 16 + s
        g0 = w * CPW

        zeros16 = jnp.zeros((16,), jnp.float32)
        ones16 = jnp.ones((16,), jnp.float32)

        def zrow(r, carry):
            for kk in range(8):
                rows[0, r, pl.ds(kk * 16, 16)] = zeros16
            return carry
        lax.fori_loop(0, CHUNK, zrow, 0)
        if with_cnt:
            for kk in range(CHUNK // 16):
                ones_v[pl.ds(kk * 16, 16)] = ones16

        slab = s * ROWS_PER_TILE
        for t in range(ROWS_PER_TILE // CHUNK):
            pltpu.async_copy(rows.at[0], acc.at[pl.ds(slab + t * CHUNK, CHUNK)],
                             sem_s)
        if with_cnt:
            for t in range(ROWS_PER_TILE // D):
                pltpu.async_copy(rows.at[0, 0],
                                 acc_cnt.at[pl.ds(slab + t * D, D)],
                                 sem_s)
        for t in range(ROWS_PER_TILE // CHUNK):
            pltpu.make_async_copy(
                rows.at[0], acc.at[pl.ds(slab, CHUNK)], sem_s).wait()
        if with_cnt:
            for t in range(ROWS_PER_TILE // D):
                pltpu.make_async_copy(
                    rows.at[0, 0], acc_cnt.at[pl.ds(slab, D)], sem_s).wait()

        plsc.subcore_barrier()

        pltpu.sync_copy(e_hbm.at[g0], idx.at[0])
        for p in range(1, 5):
            pltpu.async_copy(e_hbm.at[g0 + p], idx.at[p], sem_i)
        pltpu.async_copy(h_hbm.at[idx.at[0, 0]], rows.at[0], sem_g)
        pltpu.make_async_copy(e_hbm.at[g0], idx.at[0], sem_i).wait()
        pltpu.async_copy(h_hbm.at[idx.at[1, 0]], rows.at[1], sem_g)

        def step(j, u):
            @pl.when(j >= 2)
            def _():
                if with_rows:
                    pltpu.make_async_copy(
                        rows.at[0], acc.at[idx.at[0, 1]], sem_s).wait()
                if with_cnt:
                    pltpu.make_async_copy(
                        ones_v, acc_cnt.at[idx.at[0, 1]], sem_s).wait()

            @pl.when(j + 2 < CPW)
            def _():
                pltpu.make_async_copy(
                    e_hbm.at[g0], idx.at[0], sem_i).wait()
                pltpu.async_copy(
                    h_hbm.at[idx.at[lax.rem(j + 2, IB), 0]],
                    rows.at[(u + 2) % RB], sem_g)

            pltpu.make_async_copy(
                h_hbm.at[idx.at[0, 0]], rows.at[u], sem_g).wait()
            if with_rows:
                pltpu.async_copy(rows.at[u],
                                 acc.at[idx.at[lax.rem(j, IB), 1]], sem_s,
                                 add=True)
            if with_cnt:
                pltpu.async_copy(ones_v,
                                 acc_cnt.at[idx.at[lax.rem(j, IB), 1]], sem_s,
                                 add=True)

            @pl.when(j + 5 < CPW)
            def _():
                pltpu.async_copy(e_hbm.at[g0 + j + 5],
                                 idx.at[lax.rem(j + 5, IB)], sem_i)

        def chunk_group(jj, carry):
            for u in range(RB):
                step(jj * RB + u, u)
            return carry
        lax.fori_loop(0, CPW // RB, chunk_group, 0)

        for _ in range(2):
            if with_rows:
                pltpu.make_async_copy(
                    rows.at[0], acc.at[idx.at[0, 1]], sem_s).wait()
            if with_cnt:
                pltpu.make_async_copy(
                    ones_v, acc_cnt.at[idx.at[0, 1]], sem_s).wait()

        plsc.subcore_barrier()

        pltpu.sync_copy(acc.at[pl.ds(slab, ROWS_PER_TILE)],
                        sums_out.at[c, pl.ds(slab, ROWS_PER_TILE)])
        if with_cnt:
            pltpu.sync_copy(acc_cnt.at[pl.ds(slab, ROWS_PER_TILE)],
                            cnt_out.at[c, pl.ds(slab, ROWS_PER_TILE)])

    return agg


_sc_agg_cnt = _make_sc_agg(True)
_sc_agg_nocnt = _make_sc_agg(False)


def _tc_layer(h, sums_p, cnt_t, wa, wb, b2d):
    blk = 1000

    def body(h_ref, s_ref, c_ref, wa_ref, wb_ref, b_ref, o_ref):
        sums = s_ref[0] + s_ref[1]
        cnt = jnp.sum(c_ref[...], axis=1, keepdims=True)
        mean = sums * (1.0 / jnp.maximum(cnt, 1.0))
        acc = jnp.dot(h_ref[...], wa_ref[...], preferred_element_type=jnp.float32)
        acc = acc + jnp.dot(mean, wb_ref[...], preferred_element_type=jnp.float32)
        o_ref[...] = jnp.maximum(acc + b_ref[...], 0.0)

    return pl.pallas_call(
        body,
        grid=(N // blk,),
        in_specs=[
            pl.BlockSpec((blk, D), lambda i: (i, 0)),
            pl.BlockSpec((2, blk, D), lambda i: (0, i, 0)),
            pl.BlockSpec((blk, 2), lambda i: (i, 0)),
            pl.BlockSpec((D, D), lambda i: (0, 0)),
            pl.BlockSpec((D, D), lambda i: (0, 0)),
            pl.BlockSpec((1, D), lambda i: (0, 0)),
        ],
        out_specs=pl.BlockSpec((blk, D), lambda i: (i, 0)),
        out_shape=jax.ShapeDtypeStruct((N, D), jnp.float32),
    )(h, sums_p, cnt_t, wa, wb, b2d)


def kernel(h, edge_index, W1, b1, W2, b2):
    src = edge_index[0].astype(jnp.int32)
    dst = edge_index[1].astype(jnp.int32)

    epad = EPAD - E
    pad_ar = jnp.arange(epad, dtype=jnp.int32)
    src_p = jnp.concatenate([src, (pad_ar * 37) % N])
    dst_p = jnp.concatenate([dst, N + pad_ar % (NPAD - N)])
    edges3d = jnp.stack(
        [src_p.reshape(EPAD // CHUNK, CHUNK),
         dst_p.reshape(EPAD // CHUNK, CHUNK)], axis=1)

    sums_p, cnt_all = _sc_agg_cnt(h, edges3d)
    cnt_t = cnt_all.T
    h = _tc_layer(h, sums_p, cnt_t, W1[:D], W1[D:], b1.reshape(1, D))
    (sums_p2,) = _sc_agg_nocnt(h, edges3d)
    return _tc_layer(h, sums_p2, cnt_t, W2[:D], W2[D:], b2.reshape(1, D))


# speedup vs baseline: 1.3893x; 1.0839x over previous
import functools

import jax
import jax.numpy as jnp
from jax import lax
from jax.experimental import pallas as pl
from jax.experimental.pallas import tpu as pltpu
from jax.experimental.pallas import tpu_sc as plsc

N = 10000
E = 320000
D = 128

NPAD = 10240
CHUNK = 64
NCHUNK = E // CHUNK
NW = 32
CPW_LO = NCHUNK // NW
CPW_REM = NCHUNK - CPW_LO * NW
ROWS_PER_TILE = NPAD // 16
RB = 5
IB = 7


def _make_sc_agg(with_cnt):
    mesh = plsc.VectorSubcoreMesh(core_axis_name="c", subcore_axis_name="s")
    out_type = [jax.ShapeDtypeStruct((2, NPAD, D), jnp.float32)]
    scratch = [
        pltpu.VMEM((IB, 2, CHUNK), jnp.int32),
        pltpu.VMEM((RB, CHUNK, D), jnp.float32),
        pltpu.VMEM((CHUNK,), jnp.float32),
        pltpu.VMEM_SHARED((NPAD, D), jnp.float32),
        pltpu.VMEM_SHARED((NPAD,), jnp.float32),
        pltpu.SemaphoreType.DMA,
        pltpu.SemaphoreType.DMA,
        pltpu.SemaphoreType.DMA,
    ]
    if with_cnt:
        out_type = out_type + [jax.ShapeDtypeStruct((2, NPAD), jnp.float32)]

    @functools.partial(pl.kernel, mesh=mesh, out_type=out_type,
                       scratch_types=scratch)
    def agg(h_hbm, e_hbm, *rest):
        if with_cnt:
            (sums_out, cnt_out,
             idx, rows, ones_v, acc, acc_cnt, sem_i, sem_g, sem_s) = rest
        else:
            (sums_out,
             idx, rows, ones_v, acc, acc_cnt, sem_i, sem_g, sem_s) = rest
        c = lax.axis_index("c")
        s = lax.axis_index("s")
        w = c * 16 + s
        ncw = CPW_LO + jnp.where(w < CPW_REM, 1, 0)
        e0 = (w * CPW_LO + jnp.minimum(w, CPW_REM)) * CHUNK

        zeros16 = jnp.zeros((16,), jnp.float32)
        ones16 = jnp.ones((16,), jnp.float32)

        def zrow(r, carry):
            for kk in range(8):
                rows[0, r, pl.ds(kk * 16, 16)] = zeros16
            return carry
        lax.fori_loop(0, CHUNK, zrow, 0)
        if with_cnt:
            for kk in range(CHUNK // 16):
                ones_v[pl.ds(kk * 16, 16)] = ones16

        slab = s * ROWS_PER_TILE
        for t in range(ROWS_PER_TILE // CHUNK):
            pltpu.async_copy(rows.at[0], acc.at[pl.ds(slab + t * CHUNK, CHUNK)],
                             sem_s)
        if with_cnt:
            for t in range(ROWS_PER_TILE // D):
                pltpu.async_copy(rows.at[0, 0],
                                 acc_cnt.at[pl.ds(slab + t * D, D)],
                                 sem_s)
        for t in range(ROWS_PER_TILE // CHUNK):
            pltpu.make_async_copy(
                rows.at[0], acc.at[pl.ds(slab, CHUNK)], sem_s).wait()
        if with_cnt:
            for t in range(ROWS_PER_TILE // D):
                pltpu.make_async_copy(
                    rows.at[0, 0], acc_cnt.at[pl.ds(slab, D)], sem_s).wait()

        plsc.subcore_barrier()

        g0 = e0 // CHUNK

        def fetch_idx(g, slot):
            pltpu.async_copy(e_hbm.at[0, g0 + g], idx.at[slot, 0], sem_i)
            pltpu.async_copy(e_hbm.at[1, g0 + g], idx.at[slot, 1], sem_i)

        def wait_idx():
            for _ in range(2):
                pltpu.make_async_copy(
                    e_hbm.at[0, 0], idx.at[0, 0], sem_i).wait()

        pltpu.sync_copy(e_hbm.at[0, g0], idx.at[0, 0])
        pltpu.sync_copy(e_hbm.at[1, g0], idx.at[0, 1])
        for p in range(1, 5):
            fetch_idx(p, p)
        pltpu.async_copy(h_hbm.at[idx.at[0, 0]], rows.at[0], sem_g)
        wait_idx()
        pltpu.async_copy(h_hbm.at[idx.at[1, 0]], rows.at[1], sem_g)

        def step(j, carry):
            u = lax.rem(j, RB)
            @pl.when(j >= 2)
            def _():
                pltpu.make_async_copy(
                    rows.at[0], acc.at[idx.at[0, 1]], sem_s).wait()
                if with_cnt:
                    pltpu.make_async_copy(
                        ones_v, acc_cnt.at[idx.at[0, 1]], sem_s).wait()

            @pl.when(j + 2 < ncw)
            def _():
                wait_idx()
                pltpu.async_copy(
                    h_hbm.at[idx.at[lax.rem(j + 2, IB), 0]],
                    rows.at[lax.rem(j + 2, RB)], sem_g)

            pltpu.make_async_copy(
                h_hbm.at[idx.at[0, 0]], rows.at[u], sem_g).wait()
            pltpu.async_copy(rows.at[u],
                             acc.at[idx.at[lax.rem(j, IB), 1]], sem_s,
                             add=True)
            if with_cnt:
                pltpu.async_copy(ones_v,
                                 acc_cnt.at[idx.at[lax.rem(j, IB), 1]], sem_s,
                                 add=True)

            @pl.when(j + 5 < ncw)
            def _():
                fetch_idx(j + 5, lax.rem(j + 5, IB))
            return carry
        lax.fori_loop(0, ncw, step, 0)

        for _ in range(2):
            pltpu.make_async_copy(
                rows.at[0], acc.at[idx.at[0, 1]], sem_s).wait()
            if with_cnt:
                pltpu.make_async_copy(
                    ones_v, acc_cnt.at[idx.at[0, 1]], sem_s).wait()

        plsc.subcore_barrier()

        pltpu.sync_copy(acc.at[pl.ds(slab, ROWS_PER_TILE)],
                        sums_out.at[c, pl.ds(slab, ROWS_PER_TILE)])
        if with_cnt:
            pltpu.sync_copy(acc_cnt.at[pl.ds(slab, ROWS_PER_TILE)],
                            cnt_out.at[c, pl.ds(slab, ROWS_PER_TILE)])

    return agg


_sc_agg_cnt = _make_sc_agg(True)
_sc_agg_nocnt = _make_sc_agg(False)


def _tc_layer(h, sums_p, cnt_t, w2d, b2d):
    blk = 1000

    def body(h_ref, s_ref, c_ref, w_ref, b_ref, o_ref):
        sums = s_ref[0] + s_ref[1]
        cnt = jnp.sum(c_ref[...], axis=1, keepdims=True)
        mean = sums * (1.0 / jnp.maximum(cnt, 1.0))
        acc = jnp.dot(h_ref[...], w_ref[:D], preferred_element_type=jnp.float32)
        acc = acc + jnp.dot(mean, w_ref[D:], preferred_element_type=jnp.float32)
        o_ref[...] = jnp.maximum(acc + b_ref[...], 0.0)

    return pl.pallas_call(
        body,
        grid=(N // blk,),
        in_specs=[
            pl.BlockSpec((blk, D), lambda i: (i, 0)),
            pl.BlockSpec((2, blk, D), lambda i: (0, i, 0)),
            pl.BlockSpec((blk, 2), lambda i: (i, 0)),
            pl.BlockSpec((2 * D, D), lambda i: (0, 0)),
            pl.BlockSpec((1, D), lambda i: (0, 0)),
        ],
        out_specs=pl.BlockSpec((blk, D), lambda i: (i, 0)),
        out_shape=jax.ShapeDtypeStruct((N, D), jnp.float32),
    )(h, sums_p, cnt_t, w2d, b2d)


def kernel(h, edge_index, W1, b1, W2, b2):
    edges = edge_index.astype(jnp.int32).reshape(2, NCHUNK, CHUNK)

    sums_p, cnt_all = _sc_agg_cnt(h, edges)
    cnt_t = cnt_all.T
    h = _tc_layer(h, sums_p, cnt_t, W1, b1.reshape(1, D))
    (sums_p2,) = _sc_agg_nocnt(h, edges)
    return _tc_layer(h, sums_p2, cnt_t, W2, b2.reshape(1, D))
